# baseline (speedup 1.0000x reference)
import functools

import jax
import jax.numpy as jnp
from jax import lax
from jax.experimental import pallas as pl
from jax.experimental.pallas import tpu as pltpu
from jax.experimental.pallas import tpu_sc as plsc

N = 10000
K = 32
C = 128
L = 16
NCC = C // L
NW = 32
RPW = 320
NP = NW * RPW
CH = 2
NCH = RPW // CH
NBUF = 4


def _mm_body(x_ref, w_ref, b_ref, a_ref, t_ref):
    xb = x_ref[...]
    w = w_ref[...]
    wd = w[:C, :] - w[C:, :]
    a_ref[...] = jnp.dot(xb, wd, preferred_element_type=jnp.float32) + b_ref[...]
    t_ref[...] = jnp.dot(xb, w[C:, :], preferred_element_type=jnp.float32)


def _tc_matmul(x_pad, W, b2d):
    BLK = 1024
    return pl.pallas_call(
        _mm_body,
        grid=(NP // BLK,),
        in_specs=[
            pl.BlockSpec((BLK, C), lambda i: (i, 0)),
            pl.BlockSpec((2 * C, C), lambda i: (0, 0)),
            pl.BlockSpec((1, C), lambda i: (0, 0)),
        ],
        out_specs=[
            pl.BlockSpec((BLK, C), lambda i: (i, 0)),
            pl.BlockSpec((BLK, C), lambda i: (i, 0)),
        ],
        out_shape=[
            jax.ShapeDtypeStruct((NP, C), jnp.float32),
            jax.ShapeDtypeStruct((NP, C), jnp.float32),
        ],
    )(x_pad, W, b2d)


def _sc_body(idx_hbm, a_hbm, tab_hbm, out_hbm, idx_v, a_v, rows_v, out_v,
             *sems):
    cid = lax.axis_index("c")
    sid = lax.axis_index("s")
    wid = sid * 2 + cid
    rbase = wid * RPW
    pltpu.sync_copy(idx_hbm.at[pl.ds(rbase * K, RPW * K)], idx_v)
    pltpu.sync_copy(a_hbm.at[pl.ds(rbase, RPW)], a_v)

    def start(ck, buf):
        pltpu.make_async_copy(
            tab_hbm.at[idx_v.at[pl.ds(ck * (CH * K), CH * K)]],
            rows_v.at[buf],
            sems[buf],
        ).start()

    def wait(buf):
        pltpu.make_async_copy(
            tab_hbm.at[idx_v.at[pl.ds(0, CH * K)]],
            rows_v.at[buf],
            sems[buf],
        ).wait()

    for buf in range(NBUF):
        start(buf, buf)

    def step(ckg, carry):
        for buf in range(NBUF):
            ck = ckg * NBUF + buf
            wait(buf)
            for nloc in range(CH):
                base = nloc * K
                accs = tuple(rows_v[buf, base, pl.ds(cc * L, L)]
                             for cc in range(NCC))

                def jgrp(j0, accs, _buf=buf, _base=base):
                    for dj in range(4):
                        r = _base + j0 * 4 + dj
                        accs = tuple(
                            jnp.maximum(a, rows_v[_buf, r, pl.ds(cc * L, L)])
                            for cc, a in enumerate(accs))
                    return accs

                accs = lax.fori_loop(0, K // 4, jgrp, accs)
                row = ck * CH + nloc
                for cc in range(NCC):
                    v = accs[cc] + a_v[row, pl.ds(cc * L, L)]
                    out_v[row, pl.ds(cc * L, L)] = jnp.where(
                        v > 0.0, v, jnp.exp(v) - 1.0)

            @pl.when(ck + NBUF < NCH)
            def _(_ck=ck, _buf=buf):
                start(_ck + NBUF, _buf)

        return carry

    lax.fori_loop(0, NCH // NBUF, step, 0)
    pltpu.sync_copy(out_v, out_hbm.at[pl.ds(rbase, RPW)])


_sc_gather_max = pl.kernel(
    _sc_body,
    out_type=jax.ShapeDtypeStruct((NP, C), jnp.float32),
    mesh=plsc.VectorSubcoreMesh(core_axis_name="c", subcore_axis_name="s"),
    scratch_types=[
        pltpu.VMEM((RPW * K,), jnp.int32),
        pltpu.VMEM((RPW, C), jnp.float32),
        pltpu.VMEM((NBUF, CH * K, C), jnp.float32),
        pltpu.VMEM((RPW, C), jnp.float32),
    ] + [pltpu.SemaphoreType.DMA] * NBUF,
)


def kernel(x, edge_index, W, b):
    x2 = x[0]
    x_pad = jnp.concatenate([x2, jnp.zeros((NP - N, C), x.dtype)], axis=0)
    a_full, tab = _tc_matmul(x_pad, W, b.reshape(1, C))
    eflat = edge_index[0].reshape(N * K)
    e_pad = jnp.concatenate(
        [eflat, jnp.zeros(((NP - N) * K,), jnp.int32)], axis=0)
    out = _sc_gather_max(e_pad, a_full, tab)
    return out[:N].reshape(1, N, C)


# speedup vs baseline: 5.0705x; 5.0705x over previous
import functools

import jax
import jax.numpy as jnp
from jax import lax
from jax.experimental import pallas as pl
from jax.experimental.pallas import tpu as pltpu
from jax.experimental.pallas import tpu_sc as plsc

N = 10000
K = 32
C = 128
L = 16
NCC = C // L
NW = 32
RPW = 320
NP = NW * RPW
CH = 4
NCH = RPW // CH
NBUF = 2
SEG = NP // 16


def _mm_body(x_ref, w_ref, b_ref, a_ref, t_ref):
    xb = x_ref[...]
    w = w_ref[...]
    wd = w[:C, :] - w[C:, :]
    a_ref[...] = jnp.dot(xb, wd, preferred_element_type=jnp.float32) + b_ref[...]
    t_ref[...] = jnp.dot(xb, w[C:, :], preferred_element_type=jnp.float32)


def _tc_matmul(x_pad, W, b2d):
    BLK = 1024
    return pl.pallas_call(
        _mm_body,
        grid=(NP // BLK,),
        in_specs=[
            pl.BlockSpec((BLK, C), lambda i: (i, 0)),
            pl.BlockSpec((2 * C, C), lambda i: (0, 0)),
            pl.BlockSpec((1, C), lambda i: (0, 0)),
        ],
        out_specs=[
            pl.BlockSpec((BLK, C), lambda i: (i, 0)),
            pl.BlockSpec((BLK, C), lambda i: (i, 0)),
        ],
        out_shape=[
            jax.ShapeDtypeStruct((NP, C), jnp.float32),
            jax.ShapeDtypeStruct((NP, C), jnp.float32),
        ],
    )(x_pad, W, b2d)


def _sc_body(idx_hbm, a_hbm, tab_hbm, out_hbm, idx_v, rows_v, a_b, out_b,
             tab_sh, sem_g0, sem_g1, sem_a0, sem_a1, sem_s0, sem_s1):
    cid = lax.axis_index("c")
    sid = lax.axis_index("s")
    wid = sid * 2 + cid
    rbase = wid * RPW
    pltpu.sync_copy(tab_hbm.at[pl.ds(sid * SEG, SEG)],
                    tab_sh.at[pl.ds(sid * SEG, SEG)])
    pltpu.sync_copy(idx_hbm.at[pl.ds(rbase * K, RPW * K)], idx_v)
    plsc.subcore_barrier()
    sems_g = (sem_g0, sem_g1)
    sems_a = (sem_a0, sem_a1)
    sems_s = (sem_s0, sem_s1)

    def gstart(ck, buf):
        pltpu.make_async_copy(
            tab_sh.at[idx_v.at[pl.ds(ck * (CH * K), CH * K)]],
            rows_v.at[buf],
            sems_g[buf],
        ).start()

    def gwait(buf):
        pltpu.make_async_copy(
            tab_sh.at[idx_v.at[pl.ds(0, CH * K)]],
            rows_v.at[buf],
            sems_g[buf],
        ).wait()

    def astart(ck, buf):
        pltpu.make_async_copy(
            a_hbm.at[pl.ds(rbase + ck * CH, CH)],
            a_b.at[buf],
            sems_a[buf],
        ).start()

    def await_(buf):
        pltpu.make_async_copy(
            a_hbm.at[pl.ds(rbase, CH)],
            a_b.at[buf],
            sems_a[buf],
        ).wait()

    def sstart(ck, buf):
        pltpu.make_async_copy(
            out_b.at[buf],
            out_hbm.at[pl.ds(rbase + ck * CH, CH)],
            sems_s[buf],
        ).start()

    def swait(buf):
        pltpu.make_async_copy(
            out_b.at[buf],
            out_hbm.at[pl.ds(rbase, CH)],
            sems_s[buf],
        ).wait()

    for buf in range(NBUF):
        gstart(buf, buf)
        astart(buf, buf)

    def step(ckg, carry):
        for buf in range(NBUF):
            ck = ckg * NBUF + buf
            gwait(buf)
            await_(buf)

            @pl.when(ck >= NBUF)
            def _(_buf=buf):
                swait(_buf)

            for nloc in range(CH):
                base = nloc * K
                accs = tuple(rows_v[buf, base, pl.ds(cc * L, L)]
                             for cc in range(NCC))

                def jgrp(j0, accs, _buf=buf, _base=base):
                    for dj in range(4):
                        r = _base + j0 * 4 + dj
                        accs = tuple(
                            jnp.maximum(a, rows_v[_buf, r, pl.ds(cc * L, L)])
                            for cc, a in enumerate(accs))
                    return accs

                accs = lax.fori_loop(0, K // 4, jgrp, accs)
                for cc in range(NCC):
                    v = accs[cc] + a_b[buf, nloc, pl.ds(cc * L, L)]
                    out_b[buf, nloc, pl.ds(cc * L, L)] = jnp.where(
                        v > 0.0, v, jnp.exp(v) - 1.0)

            sstart(ck, buf)

            @pl.when(ck + NBUF < NCH)
            def _(_ck=ck, _buf=buf):
                gstart(_ck + NBUF, _buf)
                astart(_ck + NBUF, _buf)

        return carry

    lax.fori_loop(0, NCH // NBUF, step, 0)
    for buf in range(NBUF):
        swait(buf)


_sc_gather_max = pl.kernel(
    _sc_body,
    out_type=jax.ShapeDtypeStruct((NP, C), jnp.float32),
    mesh=plsc.VectorSubcoreMesh(core_axis_name="c", subcore_axis_name="s"),
    scratch_types=[
        pltpu.VMEM((RPW * K,), jnp.int32),
        pltpu.VMEM((NBUF, CH * K, C), jnp.float32),
        pltpu.VMEM((NBUF, CH, C), jnp.float32),
        pltpu.VMEM((NBUF, CH, C), jnp.float32),
        pltpu.VMEM_SHARED((NP, C), jnp.float32),
    ] + [pltpu.SemaphoreType.DMA] * 6,
)


def kernel(x, edge_index, W, b):
    x2 = x[0]
    x_pad = jnp.concatenate([x2, jnp.zeros((NP - N, C), x.dtype)], axis=0)
    a_full, tab = _tc_matmul(x_pad, W, b.reshape(1, C))
    eflat = edge_index[0].reshape(N * K)
    e_pad = jnp.concatenate(
        [eflat, jnp.zeros(((NP - N) * K,), jnp.int32)], axis=0)
    out = _sc_gather_max(e_pad, a_full, tab)
    return out[:N].reshape(1, N, C)
